# baseline (speedup 1.0000x reference)
import jax
import jax.numpy as jnp
from jax import lax
from jax.experimental import pallas as pl
from jax.experimental.pallas import tpu as pltpu
from jax.experimental.pallas import tpu_sc as plsc

T = 2048
H = 1024
E = 64
K = 8
F = 256
NG = 8
TG = 4
GS = E // NG
SCALE = 1.0
CAP = (T * K // E) * 2

TB = 512
NTB = T // TB

NC = 2
NS = 16
NWRK = NC * NS
RC = 16
DPW = (E * CAP) // NWRK
TPW = T // NWRK
GT = 4
DC = 32


def _gate_body(h_ref, wr_ref, b_ref, dst_ref, gsrc_ref, wcomb_ref, carry_ref):
    i = pl.program_id(0)

    @pl.when(i == 0)
    def _():
        carry_ref[...] = jnp.zeros_like(carry_ref)

    hb = h_ref[...]
    logits = lax.dot_general(hb, wr_ref[...], (((1,), (1,)), ((), ())),
                             preferred_element_type=jnp.float32)
    scores = jax.nn.sigmoid(logits)
    sfc = scores + b_ref[...]
    lane = lax.broadcasted_iota(jnp.int32, (TB, E), 1)
    NEGF = jnp.float32(-1e30)

    sfc3 = sfc.reshape(TB, NG, GS)
    l3 = lax.broadcasted_iota(jnp.int32, (TB, NG, GS), 2)
    m1 = jnp.max(sfc3, axis=2, keepdims=True)
    i1 = jnp.min(jnp.where(sfc3 == m1, l3, GS), axis=2, keepdims=True)
    m2 = jnp.max(jnp.where(l3 == i1, NEGF, sfc3), axis=2, keepdims=True)
    gsc = (m1 + m2)[:, :, 0]

    gl = lax.broadcasted_iota(jnp.int32, (TB, NG), 1)
    gmask8 = jnp.zeros((TB, NG), jnp.float32)
    for _ in range(TG):
        m = jnp.max(gsc, axis=1, keepdims=True)
        il = jnp.min(jnp.where(gsc == m, gl, NG), axis=1, keepdims=True)
        sel = gl == il
        gmask8 = jnp.maximum(gmask8, sel.astype(jnp.float32))
        gsc = jnp.where(sel, NEGF, gsc)
    gsel = (lax.broadcasted_iota(jnp.int32, (NG, E), 1) // GS ==
            lax.broadcasted_iota(jnp.int32, (NG, E), 0)).astype(jnp.float32)
    gmask = lax.dot_general(gmask8, gsel, (((1,), (0,)), ((), ())),
                            preferred_element_type=jnp.float32) > 0.5

    masked = jnp.where(gmask, sfc, 0.0)
    es, ws = [], []
    for _ in range(K):
        m = jnp.max(masked, axis=1, keepdims=True)
        ik = jnp.min(jnp.where(masked == m, lane, E), axis=1, keepdims=True)
        sel = lane == ik
        wk = jnp.max(jnp.where(sel, scores, NEGF), axis=1, keepdims=True)
        es.append(ik)
        ws.append(wk)
        masked = jnp.where(sel, NEGF, masked)
    wsum = ws[0]
    for k in range(1, K):
        wsum = wsum + ws[k]
    wsum = wsum + 1e-20

    toh = jnp.zeros((TB, E), jnp.float32)
    for k in range(K):
        toh = toh + (lane == es[k]).astype(jnp.float32)
    r = lax.broadcasted_iota(jnp.int32, (TB, TB), 0)
    c = lax.broadcasted_iota(jnp.int32, (TB, TB), 1)
    ltri = (r > c).astype(jnp.float32)
    cumex = lax.dot_general(ltri, toh, (((1,), (0,)), ((), ())),
                            preferred_element_type=jnp.float32)
    carry = carry_ref[...]

    lane8 = lax.broadcasted_iota(jnp.int32, (TB, K), 1)
    dst = jnp.zeros((TB, K), jnp.int32)
    gsrc = jnp.zeros((TB, K), jnp.int32)
    wcm = jnp.zeros((TB, K), jnp.float32)
    for k in range(K):
        selk = (lane == es[k]).astype(jnp.float32)
        posf = jnp.sum(selk * (cumex + carry), axis=1, keepdims=True)
        pos = posf.astype(jnp.int32)
        validf = (pos < CAP).astype(jnp.float32)
        posc = jnp.minimum(pos, CAP - 1)
        dkk = jnp.where(pos < CAP, es[k] * CAP + pos, jnp.int32(E * CAP))
        gkk = es[k] * CAP + posc
        wkk = (ws[k] / wsum) * validf
        put = lane8 == k
        dst = jnp.where(put, dkk, dst)
        gsrc = jnp.where(put, gkk, gsrc)
        wcm = jnp.where(put, wkk, wcm)
    dst_ref[...] = dst
    gsrc_ref[...] = gsrc
    wcomb_ref[...] = wcm
    carry_ref[...] = carry + jnp.sum(toh, axis=0, keepdims=True)


def _gate(h, Wr, bias):
    return pl.pallas_call(
        _gate_body,
        grid=(NTB,),
        in_specs=[
            pl.BlockSpec((TB, H), lambda i: (i, 0)),
            pl.BlockSpec((E, H), lambda i: (0, 0)),
            pl.BlockSpec((1, E), lambda i: (0, 0)),
        ],
        out_specs=[
            pl.BlockSpec((TB, K), lambda i: (i, 0)),
            pl.BlockSpec((TB, K), lambda i: (i, 0)),
            pl.BlockSpec((TB, K), lambda i: (i, 0)),
        ],
        out_shape=[
            jax.ShapeDtypeStruct((T, K), jnp.int32),
            jax.ShapeDtypeStruct((T, K), jnp.int32),
            jax.ShapeDtypeStruct((T, K), jnp.float32),
        ],
        scratch_shapes=[pltpu.VMEM((1, E), jnp.float32)],
        compiler_params=pltpu.CompilerParams(
            dimension_semantics=("arbitrary",)),
    )(h, Wr, bias.reshape(1, E))


def _shared_body(h_ref, sg_ref, su_ref, sd_ref, o_ref):
    hb = h_ref[...].astype(jnp.bfloat16)
    g = jnp.dot(hb, sg_ref[...].astype(jnp.bfloat16),
                preferred_element_type=jnp.float32)
    u = jnp.dot(hb, su_ref[...].astype(jnp.bfloat16),
                preferred_element_type=jnp.float32)
    a = (g * jax.nn.sigmoid(g) * u).astype(jnp.bfloat16)
    o_ref[...] = jnp.dot(a, sd_ref[...].astype(jnp.bfloat16),
                         preferred_element_type=jnp.float32)


def _shared(h, Sg, Su, Sd):
    HI = Sg.shape[1]
    return pl.pallas_call(
        _shared_body,
        grid=(NTB,),
        in_specs=[
            pl.BlockSpec((TB, H), lambda i: (i, 0)),
            pl.BlockSpec((H, HI), lambda i: (0, 0)),
            pl.BlockSpec((H, HI), lambda i: (0, 0)),
            pl.BlockSpec((HI, H), lambda i: (0, 0)),
        ],
        out_specs=pl.BlockSpec((TB, H), lambda i: (i, 0)),
        out_shape=jax.ShapeDtypeStruct((T, H), jnp.float32),
        compiler_params=pltpu.CompilerParams(
            dimension_semantics=("arbitrary",)),
    )(h, Sg, Su, Sd)


def _ffn_body(buf_ref, w1_ref, w3_ref, w2_ref, o_ref):
    x = buf_ref[...].astype(jnp.bfloat16)
    g = jnp.dot(x, w1_ref[0].astype(jnp.bfloat16),
                preferred_element_type=jnp.float32)
    u = jnp.dot(x, w3_ref[0].astype(jnp.bfloat16),
                preferred_element_type=jnp.float32)
    a = (g * jax.nn.sigmoid(g) * u).astype(jnp.bfloat16)
    o_ref[...] = jnp.dot(a, w2_ref[0].astype(jnp.bfloat16),
                         preferred_element_type=jnp.float32)


def _ffn(buf, W1, W3, W2):
    return pl.pallas_call(
        _ffn_body,
        grid=(E,),
        in_specs=[
            pl.BlockSpec((CAP, H), lambda e: (e, 0)),
            pl.BlockSpec((1, H, F), lambda e: (e, 0, 0)),
            pl.BlockSpec((1, H, F), lambda e: (e, 0, 0)),
            pl.BlockSpec((1, F, H), lambda e: (e, 0, 0)),
        ],
        out_specs=pl.BlockSpec((CAP, H), lambda e: (e, 0)),
        out_shape=jax.ShapeDtypeStruct((E * CAP, H), jnp.float32),
        compiler_params=pltpu.CompilerParams(
            dimension_semantics=("arbitrary",)),
    )(buf, W1, W3, W2)


def _disp_body(dstT_hbm, h_hbm, buf_hbm, rows_v, idx_vs, sem):
    wid = lax.axis_index("s") * NC + lax.axis_index("c")
    t0 = wid * TPW
    for c in range(TPW // DC):
        pltpu.sync_copy(h_hbm.at[pl.ds(t0 + c * DC, DC)], rows_v)
        cps = []
        for k in range(K):
            pltpu.sync_copy(dstT_hbm.at[k, pl.ds(t0 + c * DC, DC)], idx_vs[k])
            cps.append(pltpu.async_copy(rows_v, buf_hbm.at[idx_vs[k]], sem))
        for cp in cps:
            cp.wait()


def _dispatch(dstT, h):
    mesh = plsc.VectorSubcoreMesh(core_axis_name="c", subcore_axis_name="s")
    kfn = pl.kernel(
        _disp_body,
        out_type=jax.ShapeDtypeStruct(((E + 1) * CAP, H), jnp.float32),
        mesh=mesh,
        scratch_types=[
            pltpu.VMEM((DC, H), jnp.float32),
            [pltpu.VMEM((DC,), jnp.int32) for _ in range(K)],
            pltpu.SemaphoreType.DMA,
        ],
    )
    return kfn(dstT, h)


def _comb_body(o_hbm, gsrc_hbm, wb_hbm, sh_hbm, out_hbm,
               gidx_v, wb_v, rows_vs, sh_vs, gsems, ssems, osems):
    wid = lax.axis_index("s") * NC + lax.axis_index("c")
    t0 = wid * TPW
    pltpu.sync_copy(gsrc_hbm.at[pl.ds(t0 * K, TPW * K)], gidx_v)
    pltpu.sync_copy(wb_hbm.at[pl.ds(t0 * K, TPW * K)],
                    wb_v.at[pl.ds(0, TPW * K)])
    NGRP = TPW // GT

    def start(grp):
        b = grp % 2
        idx = gidx_v.at[pl.ds(grp * GT * K, GT * K)]
        g = pltpu.async_copy(o_hbm.at[idx], rows_vs[b], gsems[b])
        s = pltpu.async_copy(sh_hbm.at[pl.ds(t0 + grp * GT, GT)],
                             sh_vs[b], ssems[b])
        return g, s

    cps = [None, None]
    ocps = [None, None]
    cps[0] = start(0)
    for grp in range(NGRP):
        b = grp % 2
        nb = (grp + 1) % 2
        if grp + 1 < NGRP:
            if ocps[nb] is not None:
                ocps[nb].wait()
                ocps[nb] = None
            cps[nb] = start(grp + 1)
        cps[b][0].wait()
        cps[b][1].wait()
        for tt in range(GT):
            sl = grp * GT * K + tt * K
            wv16 = wb_v[pl.ds(sl, 16)]
            wvecs = [wv16[k] for k in range(K)]

            def inner(cc, carry, b=b, tt=tt, wvecs=wvecs):
                acc = sh_vs[b][tt, pl.ds(cc * 16, 16)]
                for k in range(K):
                    acc = acc + rows_vs[b][tt * K + k,
                                           pl.ds(cc * 16, 16)] * wvecs[k]
                sh_vs[b][tt, pl.ds(cc * 16, 16)] = acc
                return carry

            lax.fori_loop(0, H // 16, inner, 0)
        ocps[b] = pltpu.async_copy(sh_vs[b],
                                   out_hbm.at[pl.ds(t0 + grp * GT, GT)],
                                   osems[b])
    for b in range(2):
        if ocps[b] is not None:
            ocps[b].wait()


def _combine(o_flat, gsrc_flat, wb, shared):
    mesh = plsc.VectorSubcoreMesh(core_axis_name="c", subcore_axis_name="s")
    kfn = pl.kernel(
        _comb_body,
        out_type=jax.ShapeDtypeStruct((T, H), jnp.float32),
        mesh=mesh,
        scratch_types=[
            pltpu.VMEM((TPW * K,), jnp.int32),
            pltpu.VMEM((TPW * K + 16,), jnp.float32),
            [pltpu.VMEM((GT * K, H), jnp.float32) for _ in range(2)],
            [pltpu.VMEM((GT, H), jnp.float32) for _ in range(2)],
            [pltpu.SemaphoreType.DMA for _ in range(2)],
            [pltpu.SemaphoreType.DMA for _ in range(2)],
            [pltpu.SemaphoreType.DMA for _ in range(2)],
        ],
    )
    return kfn(o_flat, gsrc_flat, wb, shared)


def kernel(h, Wr, bias, W1, W3, W2, Sg, Su, Sd):
    dst, gsrc, wcomb = _gate(h, Wr, bias)
    buf = _dispatch(dst.T, h)
    shared = _shared(h, Sg, Su, Sd)
    o = _ffn(buf, W1, W3, W2)
    out = _combine(o, gsrc.reshape(-1), wcomb.reshape(-1), shared)
    return out


# speedup vs baseline: 1.0095x; 1.0095x over previous
import jax
import jax.numpy as jnp
from jax import lax
from jax.experimental import pallas as pl
from jax.experimental.pallas import tpu as pltpu
from jax.experimental.pallas import tpu_sc as plsc

T = 2048
H = 1024
E = 64
K = 8
F = 256
NG = 8
TG = 4
GS = E // NG
SCALE = 1.0
CAP = (T * K // E) * 2

TB = 512
NTB = T // TB

NC = 2
NS = 16
NWRK = NC * NS
RC = 16
DPW = (E * CAP) // NWRK
TPW = T // NWRK
GT = 4
DC = 32


def _gate_body(h_ref, wr_ref, b_ref, dst_ref, gsrc_ref, wcomb_ref, carry_ref):
    i = pl.program_id(0)

    @pl.when(i == 0)
    def _():
        carry_ref[...] = jnp.zeros_like(carry_ref)

    hb = h_ref[...]
    logits = lax.dot_general(hb, wr_ref[...], (((1,), (1,)), ((), ())),
                             preferred_element_type=jnp.float32)
    scores = jax.nn.sigmoid(logits)
    sfc = scores + b_ref[...]
    lane = lax.broadcasted_iota(jnp.int32, (TB, E), 1)
    NEGF = jnp.float32(-1e30)

    gscore = jnp.zeros((TB, E), jnp.float32)
    for g in range(NG):
        mg = (lane // GS) == g
        xg = jnp.where(mg, sfc, NEGF)
        m1 = jnp.max(xg, axis=1, keepdims=True)
        i1 = jnp.min(jnp.where(xg == m1, lane, E), axis=1, keepdims=True)
        m2 = jnp.max(jnp.where(lane == i1, NEGF, xg), axis=1, keepdims=True)
        gscore = gscore + jnp.where(mg, m1 + m2, 0.0)

    glane = lane // GS
    gmask = jnp.zeros((TB, E), jnp.bool_)
    gtmp = gscore
    for _ in range(TG):
        m = jnp.max(gtmp, axis=1, keepdims=True)
        il = jnp.min(jnp.where(gtmp == m, lane, E), axis=1, keepdims=True)
        sel = glane == (il // GS)
        gmask = jnp.logical_or(gmask, sel)
        gtmp = jnp.where(sel, NEGF, gtmp)

    masked = jnp.where(gmask, sfc, 0.0)
    es, ws = [], []
    for _ in range(K):
        m = jnp.max(masked, axis=1, keepdims=True)
        ik = jnp.min(jnp.where(masked == m, lane, E), axis=1, keepdims=True)
        sel = lane == ik
        wk = jnp.max(jnp.where(sel, scores, NEGF), axis=1, keepdims=True)
        es.append(ik)
        ws.append(wk)
        masked = jnp.where(sel, NEGF, masked)
    wsum = ws[0]
    for k in range(1, K):
        wsum = wsum + ws[k]
    wsum = wsum + 1e-20

    toh = jnp.zeros((TB, E), jnp.float32)
    for k in range(K):
        toh = toh + (lane == es[k]).astype(jnp.float32)
    r = lax.broadcasted_iota(jnp.int32, (TB, TB), 0)
    c = lax.broadcasted_iota(jnp.int32, (TB, TB), 1)
    ltri = (r > c).astype(jnp.float32)
    cumex = lax.dot_general(ltri, toh, (((1,), (0,)), ((), ())),
                            preferred_element_type=jnp.float32)
    carry = carry_ref[...]
    ce = cumex + carry

    lane8 = lax.broadcasted_iota(jnp.int32, (TB, K), 1)
    dst = jnp.zeros((TB, K), jnp.int32)
    gsrc = jnp.zeros((TB, K), jnp.int32)
    wcm = jnp.zeros((TB, K), jnp.float32)
    for k in range(K):
        selk = (lane == es[k]).astype(jnp.float32)
        posf = jnp.sum(selk * ce, axis=1, keepdims=True)
        pos = posf.astype(jnp.int32)
        validf = (pos < CAP).astype(jnp.float32)
        posc = jnp.minimum(pos, CAP - 1)
        dkk = jnp.where(pos < CAP, es[k] * CAP + pos, jnp.int32(E * CAP))
        gkk = es[k] * CAP + posc
        wkk = (ws[k] / wsum) * validf
        put = lane8 == k
        dst = jnp.where(put, dkk, dst)
        gsrc = jnp.where(put, gkk, gsrc)
        wcm = jnp.where(put, wkk, wcm)
    dst_ref[...] = dst
    gsrc_ref[...] = gsrc
    wcomb_ref[...] = wcm
    carry_ref[...] = carry + jnp.sum(toh, axis=0, keepdims=True)


def _gate(h, Wr, bias):
    return pl.pallas_call(
        _gate_body,
        grid=(NTB,),
        in_specs=[
            pl.BlockSpec((TB, H), lambda i: (i, 0)),
            pl.BlockSpec((E, H), lambda i: (0, 0)),
            pl.BlockSpec((1, E), lambda i: (0, 0)),
        ],
        out_specs=[
            pl.BlockSpec((TB, K), lambda i: (i, 0)),
            pl.BlockSpec((TB, K), lambda i: (i, 0)),
            pl.BlockSpec((TB, K), lambda i: (i, 0)),
        ],
        out_shape=[
            jax.ShapeDtypeStruct((T, K), jnp.int32),
            jax.ShapeDtypeStruct((T, K), jnp.int32),
            jax.ShapeDtypeStruct((T, K), jnp.float32),
        ],
        scratch_shapes=[pltpu.VMEM((1, E), jnp.float32)],
        compiler_params=pltpu.CompilerParams(
            dimension_semantics=("arbitrary",)),
    )(h, Wr, bias.reshape(1, E))


def _shared_body(h_ref, sg_ref, su_ref, sd_ref, o_ref):
    hb = h_ref[...].astype(jnp.bfloat16)
    g = jnp.dot(hb, sg_ref[...].astype(jnp.bfloat16),
                preferred_element_type=jnp.float32)
    u = jnp.dot(hb, su_ref[...].astype(jnp.bfloat16),
                preferred_element_type=jnp.float32)
    a = (g * jax.nn.sigmoid(g) * u).astype(jnp.bfloat16)
    o_ref[...] = jnp.dot(a, sd_ref[...].astype(jnp.bfloat16),
                         preferred_element_type=jnp.float32)


def _shared(h, Sg, Su, Sd):
    HI = Sg.shape[1]
    return pl.pallas_call(
        _shared_body,
        grid=(NTB,),
        in_specs=[
            pl.BlockSpec((TB, H), lambda i: (i, 0)),
            pl.BlockSpec((H, HI), lambda i: (0, 0)),
            pl.BlockSpec((H, HI), lambda i: (0, 0)),
            pl.BlockSpec((HI, H), lambda i: (0, 0)),
        ],
        out_specs=pl.BlockSpec((TB, H), lambda i: (i, 0)),
        out_shape=jax.ShapeDtypeStruct((T, H), jnp.float32),
        compiler_params=pltpu.CompilerParams(
            dimension_semantics=("arbitrary",)),
    )(h, Sg, Su, Sd)


def _ffn_body(buf_ref, w1_ref, w3_ref, w2_ref, o_ref):
    x = buf_ref[...].astype(jnp.bfloat16)
    g = jnp.dot(x, w1_ref[0].astype(jnp.bfloat16),
                preferred_element_type=jnp.float32)
    u = jnp.dot(x, w3_ref[0].astype(jnp.bfloat16),
                preferred_element_type=jnp.float32)
    a = (g * jax.nn.sigmoid(g) * u).astype(jnp.bfloat16)
    o_ref[...] = jnp.dot(a, w2_ref[0].astype(jnp.bfloat16),
                         preferred_element_type=jnp.float32)


def _ffn(buf, W1, W3, W2):
    return pl.pallas_call(
        _ffn_body,
        grid=(E,),
        in_specs=[
            pl.BlockSpec((CAP, H), lambda e: (e, 0)),
            pl.BlockSpec((1, H, F), lambda e: (e, 0, 0)),
            pl.BlockSpec((1, H, F), lambda e: (e, 0, 0)),
            pl.BlockSpec((1, F, H), lambda e: (e, 0, 0)),
        ],
        out_specs=pl.BlockSpec((CAP, H), lambda e: (e, 0)),
        out_shape=jax.ShapeDtypeStruct((E * CAP, H), jnp.float32),
        compiler_params=pltpu.CompilerParams(
            dimension_semantics=("arbitrary",)),
    )(buf, W1, W3, W2)


def _disp_body(dstT_hbm, h_hbm, buf_hbm, rows_v, idx_vs, sem):
    wid = lax.axis_index("s") * NC + lax.axis_index("c")
    t0 = wid * TPW
    for c in range(TPW // DC):
        pltpu.sync_copy(h_hbm.at[pl.ds(t0 + c * DC, DC)], rows_v)
        cps = []
        for k in range(K):
            pltpu.sync_copy(dstT_hbm.at[k, pl.ds(t0 + c * DC, DC)], idx_vs[k])
            cps.append(pltpu.async_copy(rows_v, buf_hbm.at[idx_vs[k]], sem))
        for cp in cps:
            cp.wait()


def _dispatch(dstT, h):
    mesh = plsc.VectorSubcoreMesh(core_axis_name="c", subcore_axis_name="s")
    kfn = pl.kernel(
        _disp_body,
        out_type=jax.ShapeDtypeStruct(((E + 1) * CAP, H), jnp.float32),
        mesh=mesh,
        scratch_types=[
            pltpu.VMEM((DC, H), jnp.float32),
            [pltpu.VMEM((DC,), jnp.int32) for _ in range(K)],
            pltpu.SemaphoreType.DMA,
        ],
    )
    return kfn(dstT, h)


def _comb_body(o_hbm, gsrc_hbm, wb_hbm, sh_hbm, out_hbm,
               gidx_v, wb_v, rows_vs, sh_vs, gsems, ssems, osems):
    wid = lax.axis_index("s") * NC + lax.axis_index("c")
    t0 = wid * TPW
    pltpu.sync_copy(gsrc_hbm.at[pl.ds(t0 * K, TPW * K)], gidx_v)
    pltpu.sync_copy(wb_hbm.at[pl.ds(t0 * K, TPW * K)],
                    wb_v.at[pl.ds(0, TPW * K)])
    NGRP = TPW // GT

    def start(grp):
        b = grp % 2
        idx = gidx_v.at[pl.ds(grp * GT * K, GT * K)]
        g = pltpu.async_copy(o_hbm.at[idx], rows_vs[b], gsems[b])
        s = pltpu.async_copy(sh_hbm.at[pl.ds(t0 + grp * GT, GT)],
                             sh_vs[b], ssems[b])
        return g, s

    cps = [None, None]
    ocps = [None, None]
    cps[0] = start(0)
    for grp in range(NGRP):
        b = grp % 2
        nb = (grp + 1) % 2
        if grp + 1 < NGRP:
            if ocps[nb] is not None:
                ocps[nb].wait()
                ocps[nb] = None
            cps[nb] = start(grp + 1)
        cps[b][0].wait()
        cps[b][1].wait()
        for tt in range(GT):
            sl = grp * GT * K + tt * K
            wv16 = wb_v[pl.ds(sl, 16)]
            wvecs = [wv16[k] for k in range(K)]

            def inner(cc, carry, b=b, tt=tt, wvecs=wvecs):
                acc = sh_vs[b][tt, pl.ds(cc * 16, 16)]
                for k in range(K):
                    acc = acc + rows_vs[b][tt * K + k,
                                           pl.ds(cc * 16, 16)] * wvecs[k]
                sh_vs[b][tt, pl.ds(cc * 16, 16)] = acc
                return carry

            lax.fori_loop(0, H // 16, inner, 0)
        ocps[b] = pltpu.async_copy(sh_vs[b],
                                   out_hbm.at[pl.ds(t0 + grp * GT, GT)],
                                   osems[b])
    for b in range(2):
        if ocps[b] is not None:
            ocps[b].wait()


def _combine(o_flat, gsrc_flat, wb, shared):
    mesh = plsc.VectorSubcoreMesh(core_axis_name="c", subcore_axis_name="s")
    kfn = pl.kernel(
        _comb_body,
        out_type=jax.ShapeDtypeStruct((T, H), jnp.float32),
        mesh=mesh,
        scratch_types=[
            pltpu.VMEM((TPW * K,), jnp.int32),
            pltpu.VMEM((TPW * K + 16,), jnp.float32),
            [pltpu.VMEM((GT * K, H), jnp.float32) for _ in range(2)],
            [pltpu.VMEM((GT, H), jnp.float32) for _ in range(2)],
            [pltpu.SemaphoreType.DMA for _ in range(2)],
            [pltpu.SemaphoreType.DMA for _ in range(2)],
            [pltpu.SemaphoreType.DMA for _ in range(2)],
        ],
    )
    return kfn(o_flat, gsrc_flat, wb, shared)


def kernel(h, Wr, bias, W1, W3, W2, Sg, Su, Sd):
    dst, gsrc, wcomb = _gate(h, Wr, bias)
    buf = _dispatch(dst.T, h)
    shared = _shared(h, Sg, Su, Sd)
    o = _ffn(buf, W1, W3, W2)
    out = _combine(o, gsrc.reshape(-1), wcomb.reshape(-1), shared)
    return out
